# Initial kernel scaffold; baseline (speedup 1.0000x reference)
import jax
import jax.numpy as jnp
from jax.experimental import pallas as pl


def kernel(input, edge_index1, edge_index2, unpool_rows, unpool_cols, unpool_vals, W1, b1, bn1_gamma, bn1_beta, W2, b2, bn2_gamma, bn2_beta):
    raise NotImplementedError("write your pallas kernel here")


# speedup vs baseline: 4.4481x; 4.4481x over previous
import functools

import jax
import jax.numpy as jnp
from jax import lax
from jax.experimental import pallas as pl
from jax.experimental.pallas import tpu as pltpu
from jax.experimental.pallas import tpu_sc as plsc

N1 = 10000
N2 = 20000
E = 320000
D = 128
EROWS = E // 128
NEG_SLOPE = 0.01
EPS = 1e-5
BN = 400
NDUMMY = 96


def _pad16(n):
    return ((n + 255) // 256) * 256


def _pad128(n):
    return ((n + 127) // 128) * 128


@functools.lru_cache(maxsize=None)
def _make_prop(n_nodes):
    nh = n_nodes // 2
    nh_pad = _pad128(nh + NDUMMY)
    rpt = nh_pad // 16
    full = (nh // rpt) * rpt
    tail = nh - full
    t_tail = full // rpt
    mesh = plsc.VectorSubcoreMesh(core_axis_name="c", subcore_axis_name="s")

    @functools.partial(
        pl.kernel,
        mesh=mesh,
        out_type=jax.ShapeDtypeStruct((n_nodes, D), jnp.float32),
        scratch_types=[
            pltpu.VMEM_SHARED((nh_pad, D), jnp.float32),
            pltpu.VMEM((128,), jnp.int32),
            pltpu.VMEM((1, 128), jnp.int32),
            pltpu.VMEM((128, D), jnp.float32),
            pltpu.VMEM((128, D), jnp.float32),
            pltpu.SemaphoreType.DMA,
        ],
    )
    def prop(g_hbm, srcs_hbm, dsts_hbm, s_hbm, acc, sbuf, dbuf, rows, stage, sem):
        c = lax.axis_index("c")
        s = lax.axis_index("s")

        def zrow(r, carry):
            for j in range(8):
                stage[r, pl.ds(j * 16, 16)] = jnp.zeros((16,), jnp.float32)
            return carry

        lax.fori_loop(0, 128, zrow, 0)
        off = s * rpt
        for k in range(rpt // 128):
            pltpu.sync_copy(stage, acc.at[pl.ds(off + k * 128, 128)])
        rem = rpt - (rpt // 128) * 128
        if rem:
            pltpu.sync_copy(stage.at[pl.ds(0, rem)],
                            acc.at[pl.ds(off + (rpt // 128) * 128, rem)])
        plsc.subcore_barrier()
        nmine = (EROWS - s + 15) // 16

        def body(i, carry):
            eb = pl.multiple_of((s + i * 16) * 128, 128)
            pltpu.sync_copy(srcs_hbm.at[pl.ds(eb, 128)], sbuf)
            pltpu.sync_copy(dsts_hbm.at[pl.ds(c * E + eb, 128)], dbuf.at[0])
            pltpu.async_copy(g_hbm.at[sbuf], rows, sem).wait()
            pltpu.sync_copy(rows, acc.at[dbuf.at[0]], add=True)
            return carry

        lax.fori_loop(0, nmine, body, 0)
        plsc.subcore_barrier()

        def drain(nrows):
            for k in range(nrows // 128):
                a = s * rpt + k * 128
                pltpu.sync_copy(acc.at[pl.ds(a, 128)], stage)
                pltpu.sync_copy(stage, s_hbm.at[pl.ds(c * nh + a, 128)])
            drem = nrows - (nrows // 128) * 128
            if drem:
                a = s * rpt + (nrows // 128) * 128
                pltpu.sync_copy(acc.at[pl.ds(a, drem)], stage.at[pl.ds(0, drem)])
                pltpu.sync_copy(stage.at[pl.ds(0, drem)],
                                s_hbm.at[pl.ds(c * nh + a, drem)])

        @pl.when(s < t_tail)
        def _drain_full():
            drain(rpt)

        if tail:
            @pl.when(s == t_tail)
            def _drain_tail():
                drain(tail)

    return prop


@functools.lru_cache(maxsize=None)
def _make_deg(n_pad):
    npt = n_pad // 16
    mesh = plsc.VectorSubcoreMesh(core_axis_name="c", subcore_axis_name="s")

    @functools.partial(
        pl.kernel,
        mesh=mesh,
        out_type=jax.ShapeDtypeStruct((2 * n_pad,), jnp.float32),
        scratch_types=[
            pltpu.VMEM_SHARED((n_pad,), jnp.float32),
            pltpu.VMEM((1, 128), jnp.int32),
            pltpu.VMEM((128,), jnp.float32),
            pltpu.VMEM((npt,), jnp.float32),
        ],
    )
    def deg(dsts_hbm, out_hbm, accd, cbuf, ones, stage):
        c = lax.axis_index("c")
        s = lax.axis_index("s")
        w = c * 16 + s

        def zrow(r, carry):
            stage[pl.ds(r * 16, 16)] = jnp.zeros((16,), jnp.float32)
            return carry

        lax.fori_loop(0, npt // 16, zrow, 0)
        pltpu.sync_copy(stage, accd.at[pl.ds(s * npt, npt)])
        for j in range(8):
            ones[pl.ds(j * 16, 16)] = jnp.ones((16,), jnp.float32)
        plsc.subcore_barrier()
        nmine = (EROWS - w + 31) // 32

        def body(i, carry):
            eb = pl.multiple_of((w + i * 32) * 128, 128)
            pltpu.sync_copy(dsts_hbm.at[pl.ds(eb, 128)], cbuf.at[0])
            pltpu.sync_copy(ones, accd.at[cbuf.at[0]], add=True)
            return carry

        lax.fori_loop(0, nmine, body, 0)
        plsc.subcore_barrier()
        pltpu.sync_copy(accd.at[pl.ds(s * npt, npt)], stage)
        pltpu.sync_copy(stage, out_hbm.at[pl.ds(c * n_pad + s * npt, npt)])

    return deg


@functools.lru_cache(maxsize=None)
def _make_unpool():
    nchunks = N2 // 80
    mesh = plsc.VectorSubcoreMesh(core_axis_name="c", subcore_axis_name="s")

    @functools.partial(
        pl.kernel,
        mesh=mesh,
        out_type=jax.ShapeDtypeStruct((N2, D), jnp.float32),
        scratch_types=[
            pltpu.VMEM((80,), jnp.int32),
            pltpu.VMEM((80, D), jnp.float32),
            pltpu.SemaphoreType.DMA,
        ],
    )
    def unpool(x_hbm, cols_hbm, u_hbm, cbuf, rows, sem):
        c = lax.axis_index("c")
        s = lax.axis_index("s")
        w = c * 16 + s
        nmine = (nchunks - w + 31) // 32

        def body(k, carry):
            rb = pl.multiple_of((w + k * 32) * 80, 16)
            pltpu.sync_copy(cols_hbm.at[pl.ds(rb, 80)], cbuf)
            pltpu.async_copy(x_hbm.at[cbuf], rows, sem).wait()
            pltpu.sync_copy(rows, u_hbm.at[pl.ds(rb, 80)])
            return carry

        lax.fori_loop(0, nmine, body, 0)

    return unpool


def _dis_body(deg2_ref, dis_ref, dis2n_ref):
    dsum = deg2_ref[0] + deg2_ref[1]
    good = dsum > 0
    dis = jnp.where(good, lax.rsqrt(jnp.where(good, dsum, 1.0)), 0.0)
    dis_ref[...] = dis
    dis2n_ref[...] = -(dis * dis)


def _make_dis(n):
    return pl.pallas_call(
        _dis_body,
        grid=(n // BN,),
        in_specs=[pl.BlockSpec((2, BN, 1), lambda i: (0, i, 0))],
        out_specs=[pl.BlockSpec((BN, 1), lambda i: (i, 0)),
                   pl.BlockSpec((BN, 1), lambda i: (i, 0))],
        out_shape=[jax.ShapeDtypeStruct((n, 1), jnp.float32),
                   jax.ShapeDtypeStruct((n, 1), jnp.float32)],
    )


def _scale_body(x_ref, s_ref, y_ref):
    y_ref[...] = x_ref[...] * s_ref[...]


def _make_scale(n):
    return pl.pallas_call(
        _scale_body,
        grid=(n // BN,),
        in_specs=[pl.BlockSpec((BN, D), lambda i: (i, 0)),
                  pl.BlockSpec((BN, 1), lambda i: (i, 0))],
        out_specs=pl.BlockSpec((BN, D), lambda i: (i, 0)),
        out_shape=jax.ShapeDtypeStruct((n, D), jnp.float32),
    )


def _out_body(with_stats, tx0_ref, s0_ref, s1_ref, dis_ref, wa_ref, wb_ref,
              wc_ref, b_ref, o_ref, *maybe_st):
    dis = dis_ref[...]
    tx0 = tx0_ref[...]
    tx1 = -(s0_ref[...] * dis)
    tx2 = 2.0 * (-(s1_ref[...] * dis)) - tx0
    o = jnp.dot(tx0, wa_ref[...], preferred_element_type=jnp.float32)
    o = o + jnp.dot(tx1, wb_ref[...], preferred_element_type=jnp.float32)
    o = o + jnp.dot(tx2, wc_ref[...], preferred_element_type=jnp.float32)
    o = o + b_ref[...]
    o_ref[...] = o
    if with_stats:
        st_ref = maybe_st[0]

        @pl.when(pl.program_id(0) == 0)
        def _init():
            st_ref[...] = jnp.zeros_like(st_ref)

        st_ref[...] += jnp.concatenate(
            [jnp.sum(o, axis=0, keepdims=True),
             jnp.sum(o * o, axis=0, keepdims=True)], axis=0)


def _make_out(n, with_stats):
    out_specs = [pl.BlockSpec((BN, D), lambda i: (i, 0))]
    out_shape = [jax.ShapeDtypeStruct((n, D), jnp.float32)]
    if with_stats:
        out_specs.append(pl.BlockSpec((2, D), lambda i: (0, 0)))
        out_shape.append(jax.ShapeDtypeStruct((2, D), jnp.float32))
    return pl.pallas_call(
        functools.partial(_out_body, with_stats),
        grid=(n // BN,),
        in_specs=[pl.BlockSpec((BN, D), lambda i: (i, 0)),
                  pl.BlockSpec((BN, D), lambda i: (i, 0)),
                  pl.BlockSpec((BN, D), lambda i: (i, 0)),
                  pl.BlockSpec((BN, 1), lambda i: (i, 0)),
                  pl.BlockSpec((D, D), lambda i: (0, 0)),
                  pl.BlockSpec((D, D), lambda i: (0, 0)),
                  pl.BlockSpec((D, D), lambda i: (0, 0)),
                  pl.BlockSpec((1, D), lambda i: (0, 0))],
        out_specs=out_specs if with_stats else out_specs[0],
        out_shape=out_shape if with_stats else out_shape[0],
    )


def _stats_body(x_ref, vals_ref, st_ref):
    x = x_ref[...] * vals_ref[...]

    @pl.when(pl.program_id(0) == 0)
    def _init():
        st_ref[...] = jnp.zeros_like(st_ref)

    st_ref[...] += jnp.concatenate(
        [jnp.sum(x, axis=0, keepdims=True),
         jnp.sum(x * x, axis=0, keepdims=True)], axis=0)


def _make_stats(n):
    return pl.pallas_call(
        _stats_body,
        grid=(n // BN,),
        in_specs=[pl.BlockSpec((BN, D), lambda i: (i, 0)),
                  pl.BlockSpec((BN, 1), lambda i: (i, 0))],
        out_specs=pl.BlockSpec((2, D), lambda i: (0, 0)),
        out_shape=jax.ShapeDtypeStruct((2, D), jnp.float32),
    )


def _bn_body(n, with_vals, with_g, *refs):
    refs = list(refs)
    x_ref = refs.pop(0)
    st_ref = refs.pop(0)
    gam_ref = refs.pop(0)
    bet_ref = refs.pop(0)
    vals_ref = refs.pop(0) if with_vals else None
    dis_ref = refs.pop(0) if with_g else None
    y_ref = refs.pop(0)
    g_ref = refs.pop(0) if with_g else None

    x = x_ref[...]
    if with_vals:
        x = x * vals_ref[...]
    inv_n = 1.0 / n
    mean = st_ref[0:1, :] * inv_n
    var = st_ref[1:2, :] * inv_n - mean * mean
    scale = lax.rsqrt(var + EPS) * gam_ref[...]
    y = (x - mean) * scale + bet_ref[...]
    y = jnp.where(y >= 0, y, NEG_SLOPE * y)
    y_ref[...] = y
    if with_g:
        g_ref[...] = y * dis_ref[...]


def _make_bn(n, with_vals, with_g):
    in_specs = [pl.BlockSpec((BN, D), lambda i: (i, 0)),
                pl.BlockSpec((2, D), lambda i: (0, 0)),
                pl.BlockSpec((1, D), lambda i: (0, 0)),
                pl.BlockSpec((1, D), lambda i: (0, 0))]
    if with_vals:
        in_specs.append(pl.BlockSpec((BN, 1), lambda i: (i, 0)))
    if with_g:
        in_specs.append(pl.BlockSpec((BN, 1), lambda i: (i, 0)))
    out_specs = [pl.BlockSpec((BN, D), lambda i: (i, 0))]
    out_shape = [jax.ShapeDtypeStruct((n, D), jnp.float32)]
    if with_g:
        out_specs.append(pl.BlockSpec((BN, D), lambda i: (i, 0)))
        out_shape.append(jax.ShapeDtypeStruct((n, D), jnp.float32))
    return pl.pallas_call(
        functools.partial(_bn_body, n, with_vals, with_g),
        grid=(n // BN,),
        in_specs=in_specs,
        out_specs=out_specs if with_g else out_specs[0],
        out_shape=out_shape if with_g else out_shape[0],
    )


_dis_n1 = _make_dis(N1)
_dis_n2 = _make_dis(N2)
_scale_n1 = _make_scale(N1)
_scale_n2 = _make_scale(N2)
_out_n1 = _make_out(N1, False)
_out_n2 = _make_out(N2, True)
_stats_n2 = _make_stats(N2)
_bn_unpool = _make_bn(N2, True, True)
_bn_mid = _make_bn(N2, False, True)
_bn_final = _make_bn(N2, False, False)


def _local_dsts(dst, n_nodes):
    nh = n_nodes // 2
    spread = jnp.arange(E, dtype=jnp.int32) % NDUMMY
    parts = []
    for c in (0, 1):
        mine = (dst >= c * nh) & (dst < (c + 1) * nh)
        parts.append(jnp.where(mine, dst - c * nh, nh + spread))
    return jnp.concatenate(parts)


def kernel(input, edge_index1, edge_index2, unpool_rows, unpool_cols,
           unpool_vals, W1, b1, bn1_gamma, bn1_beta, W2, b2,
           bn2_gamma, bn2_beta):
    f32 = jnp.float32
    src1, dst1 = edge_index1[0], edge_index1[1]
    src2, dst2 = edge_index2[0], edge_index2[1]
    dl1 = _local_dsts(dst1, N1)
    dl2 = _local_dsts(dst2, N2)

    _prop_n1 = _make_prop(N1)
    _prop_n2 = _make_prop(N2)
    _unpool = _make_unpool()
    deg1p = _make_deg(_pad16(N1))(dst1)
    deg2p = _make_deg(_pad16(N2))(dst2)
    deg1p = jnp.stack([deg1p[:N1], deg1p[_pad16(N1):_pad16(N1) + N1]])
    deg2p = jnp.stack([deg2p[:N2], deg2p[_pad16(N2):_pad16(N2) + N2]])
    dis1, dis2n1 = _dis_n1(deg1p.reshape(2, N1, 1))
    dis2, dis2n2 = _dis_n2(deg2p.reshape(2, N2, 1))

    g = _scale_n1(input, dis1)
    s0 = _prop_n1(g, src1, dl1)
    g1 = _scale_n1(s0, dis2n1)
    s1 = _prop_n1(g1, src1, dl1)
    x1 = _out_n1(input, s0, s1, dis1, W1[0], W1[1], W1[2], b1.reshape(1, D))

    u = _unpool(x1, unpool_cols)
    vals = unpool_vals.reshape(N2, 1)
    st = _stats_n2(u, vals)
    y, g = _bn_unpool(u, st, bn1_gamma.reshape(1, D), bn1_beta.reshape(1, D),
                      vals, dis2)

    for i in range(4):
        s0 = _prop_n2(g, src2, dl2)
        g1 = _scale_n2(s0, dis2n2)
        s1 = _prop_n2(g1, src2, dl2)
        xp, st = _out_n2(y, s0, s1, dis2, W2[i, 0], W2[i, 1], W2[i, 2],
                         b2[i].reshape(1, D))
        if i < 3:
            y, g = _bn_mid(xp, st, bn2_gamma[i].reshape(1, D),
                           bn2_beta[i].reshape(1, D), dis2)
        else:
            y = _bn_final(xp, st, bn2_gamma[i].reshape(1, D),
                          bn2_beta[i].reshape(1, D))
    return y


# speedup vs baseline: 8.8542x; 1.9905x over previous
import functools

import jax
import jax.numpy as jnp
from jax import lax
from jax.experimental import pallas as pl
from jax.experimental.pallas import tpu as pltpu
from jax.experimental.pallas import tpu_sc as plsc

N1 = 10000
N2 = 20000
E = 320000
D = 128
EROWS = E // 128
CH = 100
NCH = E // CH
DCH = 125
DNCH = E // DCH
NEG_SLOPE = 0.01
EPS = 1e-5
BN = 400
NDUMMY = 96


def _pad16(n):
    return ((n + 255) // 256) * 256


def _pad128(n):
    return ((n + 127) // 128) * 128


@functools.lru_cache(maxsize=None)
def _make_prop(n_nodes):
    nh = n_nodes // 2
    nh_pad = _pad128(nh + NDUMMY)
    rpt = nh_pad // 16
    full = (nh // rpt) * rpt
    tail = nh - full
    t_tail = full // rpt
    cpt = NCH // 16
    gpt = cpt // 8
    mesh = plsc.VectorSubcoreMesh(core_axis_name="c", subcore_axis_name="s")

    @functools.partial(
        pl.kernel,
        mesh=mesh,
        out_type=jax.ShapeDtypeStruct((n_nodes, D), jnp.float32),
        scratch_types=[
            pltpu.VMEM_SHARED((nh_pad, D), jnp.float32),
            pltpu.VMEM((16, CH), jnp.int32),
            pltpu.VMEM((16, CH), jnp.int32),
            pltpu.VMEM((CH, D), jnp.float32),
            pltpu.VMEM((CH, D), jnp.float32),
            pltpu.SemaphoreType.DMA,
            pltpu.SemaphoreType.DMA,
            pltpu.SemaphoreType.DMA,
        ],
    )
    def prop(g_hbm, srcs_hbm, dsts_hbm, s_hbm, acc, sg, dg,
             rows0, rows1, sem0, sem1, semi):
        c = lax.axis_index("c")
        s = lax.axis_index("s")
        row0 = s * cpt
        drow0 = c * NCH + row0

        def zrow(r, carry):
            for j in range(8):
                rows0[r, pl.ds(j * 16, 16)] = jnp.zeros((16,), jnp.float32)
            return carry

        lax.fori_loop(0, CH, zrow, 0)
        nz = (CH // 8) * 8
        off = s * rpt
        for k in range(rpt // nz):
            pltpu.sync_copy(rows0.at[pl.ds(0, nz)], acc.at[pl.ds(off + k * nz, nz)])
        rem = rpt - (rpt // nz) * nz
        if rem:
            pltpu.sync_copy(rows0.at[pl.ds(0, rem)],
                            acc.at[pl.ds(off + (rpt // nz) * nz, rem)])
        plsc.subcore_barrier()

        pltpu.sync_copy(srcs_hbm.at[pl.ds(row0, 8)], sg.at[pl.ds(0, 8)])
        pltpu.sync_copy(dsts_hbm.at[pl.ds(drow0, 8)], dg.at[pl.ds(0, 8)])
        pltpu.async_copy(g_hbm.at[sg.at[0]], rows0, sem0)
        pltpu.async_copy(g_hbm.at[sg.at[1]], rows1, sem1)

        def group(g, carry):
            base = 8 * (g % 2)
            nbase = 8 - base

            @pl.when(g < gpt - 1)
            def _ldnext():
                pltpu.async_copy(srcs_hbm.at[pl.ds(row0 + (g + 1) * 8, 8)],
                                 sg.at[pl.ds(nbase, 8)], semi)
                pltpu.async_copy(dsts_hbm.at[pl.ds(drow0 + (g + 1) * 8, 8)],
                                 dg.at[pl.ds(nbase, 8)], semi)

            for j in range(8):
                rows = rows0 if j % 2 == 0 else rows1
                sem = sem0 if j % 2 == 0 else sem1
                pltpu.make_async_copy(g_hbm.at[sg.at[base + j]], rows, sem).wait()
                pltpu.sync_copy(rows, acc.at[dg.at[base + j]], add=True)
                if j < 6:
                    pltpu.async_copy(g_hbm.at[sg.at[base + j + 2]], rows, sem)
                else:
                    @pl.when(g < gpt - 1)
                    def _pfn():
                        if j == 6:
                            pltpu.make_async_copy(
                                srcs_hbm.at[pl.ds(row0, 8)],
                                sg.at[pl.ds(nbase, 8)], semi).wait()
                            pltpu.make_async_copy(
                                dsts_hbm.at[pl.ds(drow0, 8)],
                                dg.at[pl.ds(nbase, 8)], semi).wait()
                        pltpu.async_copy(g_hbm.at[sg.at[nbase + (j - 6)]],
                                         rows, sem)

            return carry

        lax.fori_loop(0, gpt, group, 0)
        plsc.subcore_barrier()

        def drain(nrows):
            for k in range(nrows // nz):
                a = s * rpt + k * nz
                pltpu.sync_copy(acc.at[pl.ds(a, nz)], rows0.at[pl.ds(0, nz)])
                pltpu.sync_copy(rows0.at[pl.ds(0, nz)],
                                s_hbm.at[pl.ds(c * nh + a, nz)])
            drem = nrows - (nrows // nz) * nz
            if drem:
                a = s * rpt + (nrows // nz) * nz
                pltpu.sync_copy(acc.at[pl.ds(a, drem)], rows1.at[pl.ds(0, drem)])
                pltpu.sync_copy(rows1.at[pl.ds(0, drem)],
                                s_hbm.at[pl.ds(c * nh + a, drem)])

        @pl.when(s < t_tail)
        def _drain_full():
            drain(rpt)

        if tail:
            @pl.when(s == t_tail)
            def _drain_tail():
                drain(tail)

    return prop


@functools.lru_cache(maxsize=None)
def _make_deg(n_pad):
    npt = n_pad // 16
    cpw = DNCH // 32
    mesh = plsc.VectorSubcoreMesh(core_axis_name="c", subcore_axis_name="s")

    @functools.partial(
        pl.kernel,
        mesh=mesh,
        out_type=jax.ShapeDtypeStruct((2 * n_pad,), jnp.float32),
        scratch_types=[
            pltpu.VMEM_SHARED((n_pad,), jnp.float32),
            pltpu.VMEM((cpw, DCH), jnp.int32),
            pltpu.VMEM((128,), jnp.float32),
            pltpu.VMEM((npt,), jnp.float32),
        ],
    )
    def deg(dsts_hbm, out_hbm, accd, dall, ones, stage):
        c = lax.axis_index("c")
        s = lax.axis_index("s")
        w = c * 16 + s
        pltpu.sync_copy(dsts_hbm.at[pl.ds(w * cpw, cpw)], dall)

        def zrow(r, carry):
            stage[pl.ds(r * 16, 16)] = jnp.zeros((16,), jnp.float32)
            return carry

        lax.fori_loop(0, npt // 16, zrow, 0)
        pltpu.sync_copy(stage, accd.at[pl.ds(s * npt, npt)])
        for j in range(8):
            ones[pl.ds(j * 16, 16)] = jnp.ones((16,), jnp.float32)
        plsc.subcore_barrier()

        def body(k, carry):
            pltpu.sync_copy(ones.at[pl.ds(0, DCH)], accd.at[dall.at[k]], add=True)
            return carry

        lax.fori_loop(0, cpw, body, 0)
        plsc.subcore_barrier()
        pltpu.sync_copy(accd.at[pl.ds(s * npt, npt)], stage)
        pltpu.sync_copy(stage, out_hbm.at[pl.ds(c * n_pad + s * npt, npt)])

    return deg


@functools.lru_cache(maxsize=None)
def _make_unpool():
    nchunks = N2 // 80
    mesh = plsc.VectorSubcoreMesh(core_axis_name="c", subcore_axis_name="s")

    @functools.partial(
        pl.kernel,
        mesh=mesh,
        out_type=jax.ShapeDtypeStruct((N2, D), jnp.float32),
        scratch_types=[
            pltpu.VMEM((80,), jnp.int32),
            pltpu.VMEM((80, D), jnp.float32),
            pltpu.SemaphoreType.DMA,
        ],
    )
    def unpool(x_hbm, cols_hbm, u_hbm, cbuf, rows, sem):
        c = lax.axis_index("c")
        s = lax.axis_index("s")
        w = c * 16 + s
        nmine = (nchunks - w + 31) // 32

        def body(k, carry):
            rb = pl.multiple_of((w + k * 32) * 80, 16)
            pltpu.sync_copy(cols_hbm.at[pl.ds(rb, 80)], cbuf)
            pltpu.async_copy(x_hbm.at[cbuf], rows, sem).wait()
            pltpu.sync_copy(rows, u_hbm.at[pl.ds(rb, 80)])
            return carry

        lax.fori_loop(0, nmine, body, 0)

    return unpool


def _dis_body(deg2_ref, dis_ref, dis2n_ref):
    dsum = deg2_ref[0] + deg2_ref[1]
    good = dsum > 0
    dis = jnp.where(good, lax.rsqrt(jnp.where(good, dsum, 1.0)), 0.0)
    dis_ref[...] = dis
    dis2n_ref[...] = -(dis * dis)


def _make_dis(n):
    return pl.pallas_call(
        _dis_body,
        grid=(n // BN,),
        in_specs=[pl.BlockSpec((2, BN, 1), lambda i: (0, i, 0))],
        out_specs=[pl.BlockSpec((BN, 1), lambda i: (i, 0)),
                   pl.BlockSpec((BN, 1), lambda i: (i, 0))],
        out_shape=[jax.ShapeDtypeStruct((n, 1), jnp.float32),
                   jax.ShapeDtypeStruct((n, 1), jnp.float32)],
    )


def _scale_body(x_ref, s_ref, y_ref):
    y_ref[...] = x_ref[...] * s_ref[...]


def _make_scale(n, n_out=None):
    return pl.pallas_call(
        _scale_body,
        grid=(n // BN,),
        in_specs=[pl.BlockSpec((BN, D), lambda i: (i, 0)),
                  pl.BlockSpec((BN, 1), lambda i: (i, 0))],
        out_specs=pl.BlockSpec((BN, D), lambda i: (i, 0)),
        out_shape=jax.ShapeDtypeStruct((n_out or n, D), jnp.float32),
    )


def _out_body(with_stats, tx0_ref, s0_ref, s1_ref, dis_ref, wa_ref, wb_ref,
              wc_ref, b_ref, o_ref, *maybe_st):
    dis = dis_ref[...]
    tx0 = tx0_ref[...]
    tx1 = -(s0_ref[...] * dis)
    tx2 = 2.0 * (-(s1_ref[...] * dis)) - tx0
    o = jnp.dot(tx0, wa_ref[...], preferred_element_type=jnp.float32)
    o = o + jnp.dot(tx1, wb_ref[...], preferred_element_type=jnp.float32)
    o = o + jnp.dot(tx2, wc_ref[...], preferred_element_type=jnp.float32)
    o = o + b_ref[...]
    o_ref[...] = o
    if with_stats:
        st_ref = maybe_st[0]

        @pl.when(pl.program_id(0) == 0)
        def _init():
            st_ref[...] = jnp.zeros_like(st_ref)

        st_ref[...] += jnp.concatenate(
            [jnp.sum(o, axis=0, keepdims=True),
             jnp.sum(o * o, axis=0, keepdims=True)], axis=0)


def _make_out(n, with_stats):
    out_specs = [pl.BlockSpec((BN, D), lambda i: (i, 0))]
    out_shape = [jax.ShapeDtypeStruct((n, D), jnp.float32)]
    if with_stats:
        out_specs.append(pl.BlockSpec((2, D), lambda i: (0, 0)))
        out_shape.append(jax.ShapeDtypeStruct((2, D), jnp.float32))
    return pl.pallas_call(
        functools.partial(_out_body, with_stats),
        grid=(n // BN,),
        in_specs=[pl.BlockSpec((BN, D), lambda i: (i, 0)),
                  pl.BlockSpec((BN, D), lambda i: (i, 0)),
                  pl.BlockSpec((BN, D), lambda i: (i, 0)),
                  pl.BlockSpec((BN, 1), lambda i: (i, 0)),
                  pl.BlockSpec((D, D), lambda i: (0, 0)),
                  pl.BlockSpec((D, D), lambda i: (0, 0)),
                  pl.BlockSpec((D, D), lambda i: (0, 0)),
                  pl.BlockSpec((1, D), lambda i: (0, 0))],
        out_specs=out_specs if with_stats else out_specs[0],
        out_shape=out_shape if with_stats else out_shape[0],
    )


def _stats_body(x_ref, vals_ref, st_ref):
    x = x_ref[...] * vals_ref[...]

    @pl.when(pl.program_id(0) == 0)
    def _init():
        st_ref[...] = jnp.zeros_like(st_ref)

    st_ref[...] += jnp.concatenate(
        [jnp.sum(x, axis=0, keepdims=True),
         jnp.sum(x * x, axis=0, keepdims=True)], axis=0)


def _make_stats(n):
    return pl.pallas_call(
        _stats_body,
        grid=(n // BN,),
        in_specs=[pl.BlockSpec((BN, D), lambda i: (i, 0)),
                  pl.BlockSpec((BN, 1), lambda i: (i, 0))],
        out_specs=pl.BlockSpec((2, D), lambda i: (0, 0)),
        out_shape=jax.ShapeDtypeStruct((2, D), jnp.float32),
    )


def _bn_body(n, with_vals, with_g, *refs):
    refs = list(refs)
    x_ref = refs.pop(0)
    st_ref = refs.pop(0)
    gam_ref = refs.pop(0)
    bet_ref = refs.pop(0)
    vals_ref = refs.pop(0) if with_vals else None
    dis_ref = refs.pop(0) if with_g else None
    y_ref = refs.pop(0)
    g_ref = refs.pop(0) if with_g else None

    x = x_ref[...]
    if with_vals:
        x = x * vals_ref[...]
    inv_n = 1.0 / n
    mean = st_ref[0:1, :] * inv_n
    var = st_ref[1:2, :] * inv_n - mean * mean
    scale = lax.rsqrt(var + EPS) * gam_ref[...]
    y = (x - mean) * scale + bet_ref[...]
    y = jnp.where(y >= 0, y, NEG_SLOPE * y)
    y_ref[...] = y
    if with_g:
        g_ref[...] = y * dis_ref[...]


def _make_bn(n, with_vals, with_g):
    in_specs = [pl.BlockSpec((BN, D), lambda i: (i, 0)),
                pl.BlockSpec((2, D), lambda i: (0, 0)),
                pl.BlockSpec((1, D), lambda i: (0, 0)),
                pl.BlockSpec((1, D), lambda i: (0, 0))]
    if with_vals:
        in_specs.append(pl.BlockSpec((BN, 1), lambda i: (i, 0)))
    if with_g:
        in_specs.append(pl.BlockSpec((BN, 1), lambda i: (i, 0)))
    out_specs = [pl.BlockSpec((BN, D), lambda i: (i, 0))]
    out_shape = [jax.ShapeDtypeStruct((n, D), jnp.float32)]
    if with_g:
        out_specs.append(pl.BlockSpec((BN, D), lambda i: (i, 0)))
        out_shape.append(jax.ShapeDtypeStruct((n, D), jnp.float32))
    return pl.pallas_call(
        functools.partial(_bn_body, n, with_vals, with_g),
        grid=(n // BN,),
        in_specs=in_specs,
        out_specs=out_specs if with_g else out_specs[0],
        out_shape=out_shape if with_g else out_shape[0],
    )


_dis_n1 = _make_dis(N1)
_dis_n2 = _make_dis(N2)
_scale_n1w = _make_scale(N1, N2)
_scale_n2 = _make_scale(N2)
_out_n1 = _make_out(N1, False)
_out_n2 = _make_out(N2, True)
_stats_n2 = _make_stats(N2)
_bn_unpool = _make_bn(N2, True, True)
_bn_mid = _make_bn(N2, False, True)
_bn_final = _make_bn(N2, False, False)


def _local_dsts(dst, n_nodes):
    nh = n_nodes // 2
    spread = jnp.arange(E, dtype=jnp.int32) % NDUMMY
    parts = []
    for c in (0, 1):
        mine = (dst >= c * nh) & (dst < (c + 1) * nh)
        parts.append(jnp.where(mine, dst - c * nh, nh + spread))
    return jnp.concatenate(parts).reshape(2 * NCH, CH)


def kernel(input, edge_index1, edge_index2, unpool_rows, unpool_cols,
           unpool_vals, W1, b1, bn1_gamma, bn1_beta, W2, b2,
           bn2_gamma, bn2_beta):
    f32 = jnp.float32
    src1, dst1 = edge_index1[0], edge_index1[1]
    src2, dst2 = edge_index2[0], edge_index2[1]
    dl1 = _local_dsts(dst1, N2)
    dl2 = _local_dsts(dst2, N2)
    src1 = src1.reshape(NCH, CH)
    src2 = src2.reshape(NCH, CH)

    _prop_n2 = _make_prop(N2)
    _unpool = _make_unpool()
    deg1p = _make_deg(_pad16(N1))(dst1.reshape(DNCH, DCH))
    deg2p = _make_deg(_pad16(N2))(dst2.reshape(DNCH, DCH))
    deg1p = jnp.stack([deg1p[:N1], deg1p[_pad16(N1):_pad16(N1) + N1]])
    deg2p = jnp.stack([deg2p[:N2], deg2p[_pad16(N2):_pad16(N2) + N2]])
    dis1, dis2n1 = _dis_n1(deg1p.reshape(2, N1, 1))
    dis2, dis2n2 = _dis_n2(deg2p.reshape(2, N2, 1))

    g = _scale_n1w(input, dis1)
    s0 = _prop_n2(g, src1, dl1)
    g1 = _scale_n1w(s0, dis2n1)
    s1 = _prop_n2(g1, src1, dl1)
    x1 = _out_n1(input, s0, s1, dis1, W1[0], W1[1], W1[2], b1.reshape(1, D))

    u = _unpool(x1, unpool_cols)
    vals = unpool_vals.reshape(N2, 1)
    st = _stats_n2(u, vals)
    y, g = _bn_unpool(u, st, bn1_gamma.reshape(1, D), bn1_beta.reshape(1, D),
                      vals, dis2)

    for i in range(4):
        s0 = _prop_n2(g, src2, dl2)
        g1 = _scale_n2(s0, dis2n2)
        s1 = _prop_n2(g1, src2, dl2)
        xp, st = _out_n2(y, s0, s1, dis2, W2[i, 0], W2[i, 1], W2[i, 2],
                         b2[i].reshape(1, D))
        if i < 3:
            y, g = _bn_mid(xp, st, bn2_gamma[i].reshape(1, D),
                           bn2_beta[i].reshape(1, D), dis2)
        else:
            y = _bn_final(xp, st, bn2_gamma[i].reshape(1, D),
                          bn2_beta[i].reshape(1, D))
    return y


# speedup vs baseline: 9.2693x; 1.0469x over previous
import functools

import jax
import jax.numpy as jnp
from jax import lax
from jax.experimental import pallas as pl
from jax.experimental.pallas import tpu as pltpu
from jax.experimental.pallas import tpu_sc as plsc

N1 = 10000
N2 = 20000
E = 320000
D = 128
EROWS = E // 128
CH = 50
NCH = E // CH
DCH = 125
DNCH = E // DCH
NEG_SLOPE = 0.01
EPS = 1e-5
BN = 400
NDUMMY = 96


def _pad16(n):
    return ((n + 255) // 256) * 256


def _pad128(n):
    return ((n + 127) // 128) * 128


@functools.lru_cache(maxsize=None)
def _make_prop(n_nodes):
    nh = n_nodes // 2
    nh_pad = _pad128(nh + NDUMMY)
    rpt = nh_pad // 16
    full = (nh // rpt) * rpt
    tail = nh - full
    t_tail = full // rpt
    cpt = NCH // 16
    gpt = cpt // 8
    mesh = plsc.VectorSubcoreMesh(core_axis_name="c", subcore_axis_name="s")

    @functools.partial(
        pl.kernel,
        mesh=mesh,
        out_type=jax.ShapeDtypeStruct((n_nodes, D), jnp.float32),
        scratch_types=[
            pltpu.VMEM_SHARED((nh_pad, D), jnp.float32),
            pltpu.VMEM((16, CH), jnp.int32),
            pltpu.VMEM((16, CH), jnp.int32),
            pltpu.VMEM((CH, D), jnp.float32),
            pltpu.VMEM((CH, D), jnp.float32),
            pltpu.VMEM((CH, D), jnp.float32),
            pltpu.VMEM((CH, D), jnp.float32),
            pltpu.SemaphoreType.DMA,
            pltpu.SemaphoreType.DMA,
            pltpu.SemaphoreType.DMA,
            pltpu.SemaphoreType.DMA,
            pltpu.SemaphoreType.DMA,
            pltpu.SemaphoreType.DMA,
            pltpu.SemaphoreType.DMA,
            pltpu.SemaphoreType.DMA,
            pltpu.SemaphoreType.DMA,
        ],
    )
    def prop(g_hbm, srcs_hbm, dsts_hbm, s_hbm, acc, sg, dg,
             rows0, rows1, rows2, rows3,
             gs0, gs1, gs2, gs3, ss0, ss1, ss2, ss3, semi):
        c = lax.axis_index("c")
        s = lax.axis_index("s")
        row0 = s * cpt
        drow0 = c * NCH + row0

        R = [rows0, rows1, rows2, rows3]
        GS = [gs0, gs1, gs2, gs3]
        SS = [ss0, ss1, ss2, ss3]

        def zrow(r, carry):
            for j in range(8):
                rows0[r, pl.ds(j * 16, 16)] = jnp.zeros((16,), jnp.float32)
            return carry

        lax.fori_loop(0, CH, zrow, 0)
        nz = (CH // 8) * 8
        off = s * rpt
        for k in range(rpt // nz):
            pltpu.sync_copy(rows0.at[pl.ds(0, nz)], acc.at[pl.ds(off + k * nz, nz)])
        rem = rpt - (rpt // nz) * nz
        if rem:
            pltpu.sync_copy(rows0.at[pl.ds(0, rem)],
                            acc.at[pl.ds(off + (rpt // nz) * nz, rem)])
        plsc.subcore_barrier()

        pltpu.sync_copy(srcs_hbm.at[pl.ds(row0, 8)], sg.at[pl.ds(0, 8)])
        pltpu.sync_copy(dsts_hbm.at[pl.ds(drow0, 8)], dg.at[pl.ds(0, 8)])
        for j in range(3):
            pltpu.async_copy(g_hbm.at[sg.at[j]], R[j], GS[j])

        def group(g, carry):
            base = 8 * (g % 2)
            nbase = 8 - base

            for j in range(8):
                b = j % 4
                pltpu.make_async_copy(g_hbm.at[sg.at[base + j]], R[b],
                                      GS[b]).wait()
                pltpu.async_copy(R[b], acc.at[dg.at[base + j]], SS[b], add=True)
                pb = (j + 3) % 4

                def _wait_prev_scatter():
                    pltpu.make_async_copy(R[pb], acc.at[dg.at[0]], SS[pb]).wait()

                if j < 5:
                    if j == 0:
                        @pl.when(g > 0)
                        def _w0():
                            _wait_prev_scatter()
                    else:
                        _wait_prev_scatter()
                    pltpu.async_copy(g_hbm.at[sg.at[base + j + 3]], R[pb], GS[pb])
                    if j == 1:
                        @pl.when(g < gpt - 1)
                        def _ldnext():
                            pltpu.async_copy(
                                srcs_hbm.at[pl.ds(row0 + (g + 1) * 8, 8)],
                                sg.at[pl.ds(nbase, 8)], semi)
                            pltpu.async_copy(
                                dsts_hbm.at[pl.ds(drow0 + (g + 1) * 8, 8)],
                                dg.at[pl.ds(nbase, 8)], semi)
                else:
                    @pl.when(g < gpt - 1)
                    def _pfn():
                        if j == 5:
                            pltpu.make_async_copy(
                                srcs_hbm.at[pl.ds(row0, 8)],
                                sg.at[pl.ds(nbase, 8)], semi).wait()
                            pltpu.make_async_copy(
                                dsts_hbm.at[pl.ds(drow0, 8)],
                                dg.at[pl.ds(nbase, 8)], semi).wait()
                        _wait_prev_scatter()
                        pltpu.async_copy(g_hbm.at[sg.at[nbase + (j - 5)]],
                                         R[pb], GS[pb])

            return carry

        lax.fori_loop(0, gpt, group, 0)
        for b in range(4):
            pltpu.make_async_copy(R[b], acc.at[dg.at[0]], SS[b]).wait()
        plsc.subcore_barrier()

        def drain(nrows):
            for k in range(nrows // nz):
                a = s * rpt + k * nz
                pltpu.sync_copy(acc.at[pl.ds(a, nz)], rows0.at[pl.ds(0, nz)])
                pltpu.sync_copy(rows0.at[pl.ds(0, nz)],
                                s_hbm.at[pl.ds(c * nh + a, nz)])
            drem = nrows - (nrows // nz) * nz
            if drem:
                a = s * rpt + (nrows // nz) * nz
                pltpu.sync_copy(acc.at[pl.ds(a, drem)], rows1.at[pl.ds(0, drem)])
                pltpu.sync_copy(rows1.at[pl.ds(0, drem)],
                                s_hbm.at[pl.ds(c * nh + a, drem)])

        @pl.when(s < t_tail)
        def _drain_full():
            drain(rpt)

        if tail:
            @pl.when(s == t_tail)
            def _drain_tail():
                drain(tail)

    return prop


@functools.lru_cache(maxsize=None)
def _make_deg(n_pad):
    npt = n_pad // 16
    cpw = DNCH // 32
    mesh = plsc.VectorSubcoreMesh(core_axis_name="c", subcore_axis_name="s")

    @functools.partial(
        pl.kernel,
        mesh=mesh,
        out_type=jax.ShapeDtypeStruct((2 * n_pad,), jnp.float32),
        scratch_types=[
            pltpu.VMEM_SHARED((n_pad,), jnp.float32),
            pltpu.VMEM((cpw, DCH), jnp.int32),
            pltpu.VMEM((128,), jnp.float32),
            pltpu.VMEM((npt,), jnp.float32),
        ],
    )
    def deg(dsts_hbm, out_hbm, accd, dall, ones, stage):
        c = lax.axis_index("c")
        s = lax.axis_index("s")
        w = c * 16 + s
        pltpu.sync_copy(dsts_hbm.at[pl.ds(w * cpw, cpw)], dall)

        def zrow(r, carry):
            stage[pl.ds(r * 16, 16)] = jnp.zeros((16,), jnp.float32)
            return carry

        lax.fori_loop(0, npt // 16, zrow, 0)
        pltpu.sync_copy(stage, accd.at[pl.ds(s * npt, npt)])
        for j in range(8):
            ones[pl.ds(j * 16, 16)] = jnp.ones((16,), jnp.float32)
        plsc.subcore_barrier()

        def body(k, carry):
            pltpu.sync_copy(ones.at[pl.ds(0, DCH)], accd.at[dall.at[k]], add=True)
            return carry

        lax.fori_loop(0, cpw, body, 0)
        plsc.subcore_barrier()
        pltpu.sync_copy(accd.at[pl.ds(s * npt, npt)], stage)
        pltpu.sync_copy(stage, out_hbm.at[pl.ds(c * n_pad + s * npt, npt)])

    return deg


@functools.lru_cache(maxsize=None)
def _make_unpool():
    nchunks = N2 // 80
    mesh = plsc.VectorSubcoreMesh(core_axis_name="c", subcore_axis_name="s")

    @functools.partial(
        pl.kernel,
        mesh=mesh,
        out_type=jax.ShapeDtypeStruct((N2, D), jnp.float32),
        scratch_types=[
            pltpu.VMEM((80,), jnp.int32),
            pltpu.VMEM((80, D), jnp.float32),
            pltpu.SemaphoreType.DMA,
        ],
    )
    def unpool(x_hbm, cols_hbm, u_hbm, cbuf, rows, sem):
        c = lax.axis_index("c")
        s = lax.axis_index("s")
        w = c * 16 + s
        nmine = (nchunks - w + 31) // 32

        def body(k, carry):
            rb = pl.multiple_of((w + k * 32) * 80, 16)
            pltpu.sync_copy(cols_hbm.at[pl.ds(rb, 80)], cbuf)
            pltpu.async_copy(x_hbm.at[cbuf], rows, sem).wait()
            pltpu.sync_copy(rows, u_hbm.at[pl.ds(rb, 80)])
            return carry

        lax.fori_loop(0, nmine, body, 0)

    return unpool


def _dis_body(deg2_ref, dis_ref, dis2n_ref):
    dsum = deg2_ref[0] + deg2_ref[1]
    good = dsum > 0
    dis = jnp.where(good, lax.rsqrt(jnp.where(good, dsum, 1.0)), 0.0)
    dis_ref[...] = dis
    dis2n_ref[...] = -(dis * dis)


def _make_dis(n):
    return pl.pallas_call(
        _dis_body,
        grid=(n // BN,),
        in_specs=[pl.BlockSpec((2, BN, 1), lambda i: (0, i, 0))],
        out_specs=[pl.BlockSpec((BN, 1), lambda i: (i, 0)),
                   pl.BlockSpec((BN, 1), lambda i: (i, 0))],
        out_shape=[jax.ShapeDtypeStruct((n, 1), jnp.float32),
                   jax.ShapeDtypeStruct((n, 1), jnp.float32)],
    )


def _scale_body(x_ref, s_ref, y_ref):
    y_ref[...] = x_ref[...] * s_ref[...]


def _make_scale(n, n_out=None):
    return pl.pallas_call(
        _scale_body,
        grid=(n // BN,),
        in_specs=[pl.BlockSpec((BN, D), lambda i: (i, 0)),
                  pl.BlockSpec((BN, 1), lambda i: (i, 0))],
        out_specs=pl.BlockSpec((BN, D), lambda i: (i, 0)),
        out_shape=jax.ShapeDtypeStruct((n_out or n, D), jnp.float32),
    )


def _out_body(with_stats, tx0_ref, s0_ref, s1_ref, dis_ref, wa_ref, wb_ref,
              wc_ref, b_ref, o_ref, *maybe_st):
    dis = dis_ref[...]
    tx0 = tx0_ref[...]
    tx1 = -(s0_ref[...] * dis)
    tx2 = 2.0 * (-(s1_ref[...] * dis)) - tx0
    o = jnp.dot(tx0, wa_ref[...], preferred_element_type=jnp.float32)
    o = o + jnp.dot(tx1, wb_ref[...], preferred_element_type=jnp.float32)
    o = o + jnp.dot(tx2, wc_ref[...], preferred_element_type=jnp.float32)
    o = o + b_ref[...]
    o_ref[...] = o
    if with_stats:
        st_ref = maybe_st[0]

        @pl.when(pl.program_id(0) == 0)
        def _init():
            st_ref[...] = jnp.zeros_like(st_ref)

        st_ref[...] += jnp.concatenate(
            [jnp.sum(o, axis=0, keepdims=True),
             jnp.sum(o * o, axis=0, keepdims=True)], axis=0)


def _make_out(n, with_stats):
    out_specs = [pl.BlockSpec((BN, D), lambda i: (i, 0))]
    out_shape = [jax.ShapeDtypeStruct((n, D), jnp.float32)]
    if with_stats:
        out_specs.append(pl.BlockSpec((2, D), lambda i: (0, 0)))
        out_shape.append(jax.ShapeDtypeStruct((2, D), jnp.float32))
    return pl.pallas_call(
        functools.partial(_out_body, with_stats),
        grid=(n // BN,),
        in_specs=[pl.BlockSpec((BN, D), lambda i: (i, 0)),
                  pl.BlockSpec((BN, D), lambda i: (i, 0)),
                  pl.BlockSpec((BN, D), lambda i: (i, 0)),
                  pl.BlockSpec((BN, 1), lambda i: (i, 0)),
                  pl.BlockSpec((D, D), lambda i: (0, 0)),
                  pl.BlockSpec((D, D), lambda i: (0, 0)),
                  pl.BlockSpec((D, D), lambda i: (0, 0)),
                  pl.BlockSpec((1, D), lambda i: (0, 0))],
        out_specs=out_specs if with_stats else out_specs[0],
        out_shape=out_shape if with_stats else out_shape[0],
    )


def _stats_body(x_ref, vals_ref, st_ref):
    x = x_ref[...] * vals_ref[...]

    @pl.when(pl.program_id(0) == 0)
    def _init():
        st_ref[...] = jnp.zeros_like(st_ref)

    st_ref[...] += jnp.concatenate(
        [jnp.sum(x, axis=0, keepdims=True),
         jnp.sum(x * x, axis=0, keepdims=True)], axis=0)


def _make_stats(n):
    return pl.pallas_call(
        _stats_body,
        grid=(n // BN,),
        in_specs=[pl.BlockSpec((BN, D), lambda i: (i, 0)),
                  pl.BlockSpec((BN, 1), lambda i: (i, 0))],
        out_specs=pl.BlockSpec((2, D), lambda i: (0, 0)),
        out_shape=jax.ShapeDtypeStruct((2, D), jnp.float32),
    )


def _bn_body(n, with_vals, with_g, *refs):
    refs = list(refs)
    x_ref = refs.pop(0)
    st_ref = refs.pop(0)
    gam_ref = refs.pop(0)
    bet_ref = refs.pop(0)
    vals_ref = refs.pop(0) if with_vals else None
    dis_ref = refs.pop(0) if with_g else None
    y_ref = refs.pop(0)
    g_ref = refs.pop(0) if with_g else None

    x = x_ref[...]
    if with_vals:
        x = x * vals_ref[...]
    inv_n = 1.0 / n
    mean = st_ref[0:1, :] * inv_n
    var = st_ref[1:2, :] * inv_n - mean * mean
    scale = lax.rsqrt(var + EPS) * gam_ref[...]
    y = (x - mean) * scale + bet_ref[...]
    y = jnp.where(y >= 0, y, NEG_SLOPE * y)
    y_ref[...] = y
    if with_g:
        g_ref[...] = y * dis_ref[...]


def _make_bn(n, with_vals, with_g):
    in_specs = [pl.BlockSpec((BN, D), lambda i: (i, 0)),
                pl.BlockSpec((2, D), lambda i: (0, 0)),
                pl.BlockSpec((1, D), lambda i: (0, 0)),
                pl.BlockSpec((1, D), lambda i: (0, 0))]
    if with_vals:
        in_specs.append(pl.BlockSpec((BN, 1), lambda i: (i, 0)))
    if with_g:
        in_specs.append(pl.BlockSpec((BN, 1), lambda i: (i, 0)))
    out_specs = [pl.BlockSpec((BN, D), lambda i: (i, 0))]
    out_shape = [jax.ShapeDtypeStruct((n, D), jnp.float32)]
    if with_g:
        out_specs.append(pl.BlockSpec((BN, D), lambda i: (i, 0)))
        out_shape.append(jax.ShapeDtypeStruct((n, D), jnp.float32))
    return pl.pallas_call(
        functools.partial(_bn_body, n, with_vals, with_g),
        grid=(n // BN,),
        in_specs=in_specs,
        out_specs=out_specs if with_g else out_specs[0],
        out_shape=out_shape if with_g else out_shape[0],
    )


_dis_n1 = _make_dis(N1)
_dis_n2 = _make_dis(N2)
_scale_n1w = _make_scale(N1, N2)
_scale_n2 = _make_scale(N2)
_out_n1 = _make_out(N1, False)
_out_n2 = _make_out(N2, True)
_stats_n2 = _make_stats(N2)
_bn_unpool = _make_bn(N2, True, True)
_bn_mid = _make_bn(N2, False, True)
_bn_final = _make_bn(N2, False, False)


def _local_dsts(dst, n_nodes):
    nh = n_nodes // 2
    spread = jnp.arange(E, dtype=jnp.int32) % NDUMMY
    parts = []
    for c in (0, 1):
        mine = (dst >= c * nh) & (dst < (c + 1) * nh)
        parts.append(jnp.where(mine, dst - c * nh, nh + spread))
    return jnp.concatenate(parts).reshape(2 * NCH, CH)


def kernel(input, edge_index1, edge_index2, unpool_rows, unpool_cols,
           unpool_vals, W1, b1, bn1_gamma, bn1_beta, W2, b2,
           bn2_gamma, bn2_beta):
    f32 = jnp.float32
    src1, dst1 = edge_index1[0], edge_index1[1]
    src2, dst2 = edge_index2[0], edge_index2[1]
    dl1 = _local_dsts(dst1, N2)
    dl2 = _local_dsts(dst2, N2)
    src1 = src1.reshape(NCH, CH)
    src2 = src2.reshape(NCH, CH)

    _prop_n2 = _make_prop(N2)
    _unpool = _make_unpool()
    deg1p = _make_deg(_pad16(N1))(dst1.reshape(DNCH, DCH))
    deg2p = _make_deg(_pad16(N2))(dst2.reshape(DNCH, DCH))
    deg1p = jnp.stack([deg1p[:N1], deg1p[_pad16(N1):_pad16(N1) + N1]])
    deg2p = jnp.stack([deg2p[:N2], deg2p[_pad16(N2):_pad16(N2) + N2]])
    dis1, dis2n1 = _dis_n1(deg1p.reshape(2, N1, 1))
    dis2, dis2n2 = _dis_n2(deg2p.reshape(2, N2, 1))

    g = _scale_n1w(input, dis1)
    s0 = _prop_n2(g, src1, dl1)
    g1 = _scale_n1w(s0, dis2n1)
    s1 = _prop_n2(g1, src1, dl1)
    x1 = _out_n1(input, s0, s1, dis1, W1[0], W1[1], W1[2], b1.reshape(1, D))

    u = _unpool(x1, unpool_cols)
    vals = unpool_vals.reshape(N2, 1)
    st = _stats_n2(u, vals)
    y, g = _bn_unpool(u, st, bn1_gamma.reshape(1, D), bn1_beta.reshape(1, D),
                      vals, dis2)

    for i in range(4):
        s0 = _prop_n2(g, src2, dl2)
        g1 = _scale_n2(s0, dis2n2)
        s1 = _prop_n2(g1, src2, dl2)
        xp, st = _out_n2(y, s0, s1, dis2, W2[i, 0], W2[i, 1], W2[i, 2],
                         b2[i].reshape(1, D))
        if i < 3:
            y, g = _bn_mid(xp, st, bn2_gamma[i].reshape(1, D),
                           bn2_beta[i].reshape(1, D), dis2)
        else:
            y = _bn_final(xp, st, bn2_gamma[i].reshape(1, D),
                          bn2_beta[i].reshape(1, D))
    return y


# speedup vs baseline: 9.2709x; 1.0002x over previous
import functools

import jax
import jax.numpy as jnp
from jax import lax
from jax.experimental import pallas as pl
from jax.experimental.pallas import tpu as pltpu
from jax.experimental.pallas import tpu_sc as plsc

N1 = 10000
N2 = 20000
E = 320000
D = 128
EROWS = E // 128
CH = 50
NCH = E // CH
DCH = 125
DNCH = E // DCH
NEG_SLOPE = 0.01
EPS = 1e-5
BN = 400
NDUMMY = 96


def _pad16(n):
    return ((n + 255) // 256) * 256


def _pad128(n):
    return ((n + 127) // 128) * 128


@functools.lru_cache(maxsize=None)
def _make_prop(n_nodes):
    nh = n_nodes // 2
    nh_pad = _pad128(nh + NDUMMY)
    rpt = nh_pad // 16
    full = (nh // rpt) * rpt
    tail = nh - full
    t_tail = full // rpt
    cpt = NCH // 16
    gpt = cpt // 8
    mesh = plsc.VectorSubcoreMesh(core_axis_name="c", subcore_axis_name="s")

    @functools.partial(
        pl.kernel,
        mesh=mesh,
        out_type=jax.ShapeDtypeStruct((n_nodes, D), jnp.float32),
        scratch_types=[
            pltpu.VMEM_SHARED((nh_pad, D), jnp.float32),
            pltpu.VMEM((16, CH), jnp.int32),
            pltpu.VMEM((16, CH), jnp.int32),
            pltpu.VMEM((CH, D), jnp.float32),
            pltpu.VMEM((CH, D), jnp.float32),
            pltpu.VMEM((CH, D), jnp.float32),
            pltpu.VMEM((CH, D), jnp.float32),
            pltpu.SemaphoreType.DMA,
            pltpu.SemaphoreType.DMA,
            pltpu.SemaphoreType.DMA,
            pltpu.SemaphoreType.DMA,
            pltpu.SemaphoreType.DMA,
            pltpu.SemaphoreType.DMA,
            pltpu.SemaphoreType.DMA,
            pltpu.SemaphoreType.DMA,
            pltpu.SemaphoreType.DMA,
        ],
    )
    def prop(g_hbm, srcs_hbm, dsts_hbm, s_hbm, acc, sg, dg,
             rows0, rows1, rows2, rows3,
             gs0, gs1, gs2, gs3, ss0, ss1, ss2, ss3, semi):
        c = lax.axis_index("c")
        s = lax.axis_index("s")
        row0 = s * cpt
        drow0 = c * NCH + row0

        R = [rows0, rows1, rows2, rows3]
        GS = [gs0, gs1, gs2, gs3]
        SS = [ss0, ss1, ss2, ss3]

        def zrow(r, carry):
            for j in range(8):
                rows0[r, pl.ds(j * 16, 16)] = jnp.zeros((16,), jnp.float32)
            return carry

        lax.fori_loop(0, CH, zrow, 0)
        nz = (CH // 8) * 8
        off = s * rpt
        for k in range(rpt // nz):
            pltpu.sync_copy(rows0.at[pl.ds(0, nz)], acc.at[pl.ds(off + k * nz, nz)])
        rem = rpt - (rpt // nz) * nz
        if rem:
            pltpu.sync_copy(rows0.at[pl.ds(0, rem)],
                            acc.at[pl.ds(off + (rpt // nz) * nz, rem)])
        plsc.subcore_barrier()

        pltpu.sync_copy(srcs_hbm.at[pl.ds(row0, 8)], sg.at[pl.ds(0, 8)])
        pltpu.sync_copy(dsts_hbm.at[pl.ds(drow0, 8)], dg.at[pl.ds(0, 8)])
        for j in range(3):
            pltpu.async_copy(g_hbm.at[sg.at[j]], R[j], GS[j])

        def group(g, carry):
            base = 8 * (g % 2)
            nbase = 8 - base

            for j in range(8):
                b = j % 4
                pltpu.make_async_copy(g_hbm.at[sg.at[base + j]], R[b],
                                      GS[b]).wait()
                pltpu.async_copy(R[b], acc.at[dg.at[base + j]], SS[b], add=True)
                pb = (j + 3) % 4

                def _wait_prev_scatter():
                    pltpu.make_async_copy(R[pb], acc.at[dg.at[0]], SS[pb]).wait()

                if j < 5:
                    if j == 0:
                        @pl.when(g > 0)
                        def _w0():
                            _wait_prev_scatter()
                    else:
                        _wait_prev_scatter()
                    pltpu.async_copy(g_hbm.at[sg.at[base + j + 3]], R[pb], GS[pb])
                    if j == 1:
                        @pl.when(g < gpt - 1)
                        def _ldnext():
                            pltpu.async_copy(
                                srcs_hbm.at[pl.ds(row0 + (g + 1) * 8, 8)],
                                sg.at[pl.ds(nbase, 8)], semi)
                            pltpu.async_copy(
                                dsts_hbm.at[pl.ds(drow0 + (g + 1) * 8, 8)],
                                dg.at[pl.ds(nbase, 8)], semi)
                else:
                    @pl.when(g < gpt - 1)
                    def _pfn():
                        if j == 5:
                            pltpu.make_async_copy(
                                srcs_hbm.at[pl.ds(row0, 8)],
                                sg.at[pl.ds(nbase, 8)], semi).wait()
                            pltpu.make_async_copy(
                                dsts_hbm.at[pl.ds(drow0, 8)],
                                dg.at[pl.ds(nbase, 8)], semi).wait()
                        _wait_prev_scatter()
                        pltpu.async_copy(g_hbm.at[sg.at[nbase + (j - 5)]],
                                         R[pb], GS[pb])

            return carry

        lax.fori_loop(0, gpt, group, 0)
        for b in range(4):
            pltpu.make_async_copy(R[b], acc.at[dg.at[0]], SS[b]).wait()
        plsc.subcore_barrier()

        def drain(nrows):
            for k in range(nrows // nz):
                a = s * rpt + k * nz
                pltpu.sync_copy(acc.at[pl.ds(a, nz)], rows0.at[pl.ds(0, nz)])
                pltpu.sync_copy(rows0.at[pl.ds(0, nz)],
                                s_hbm.at[pl.ds(c * nh + a, nz)])
            drem = nrows - (nrows // nz) * nz
            if drem:
                a = s * rpt + (nrows // nz) * nz
                pltpu.sync_copy(acc.at[pl.ds(a, drem)], rows1.at[pl.ds(0, drem)])
                pltpu.sync_copy(rows1.at[pl.ds(0, drem)],
                                s_hbm.at[pl.ds(c * nh + a, drem)])

        @pl.when(s < t_tail)
        def _drain_full():
            drain(rpt)

        if tail:
            @pl.when(s == t_tail)
            def _drain_tail():
                drain(tail)

    return prop


@functools.lru_cache(maxsize=None)
def _make_deg(n_pad):
    npt = n_pad // 16
    cpw = DNCH // 32
    mesh = plsc.VectorSubcoreMesh(core_axis_name="c", subcore_axis_name="s")

    @functools.partial(
        pl.kernel,
        mesh=mesh,
        out_type=jax.ShapeDtypeStruct((2 * n_pad,), jnp.float32),
        scratch_types=[
            pltpu.VMEM_SHARED((n_pad,), jnp.float32),
            pltpu.VMEM((cpw, DCH), jnp.int32),
            pltpu.VMEM((128,), jnp.float32),
            pltpu.VMEM((npt,), jnp.float32),
            pltpu.SemaphoreType.DMA,
        ],
    )
    def deg(dsts_hbm, out_hbm, accd, dall, ones, stage, semd):
        c = lax.axis_index("c")
        s = lax.axis_index("s")
        w = c * 16 + s
        pltpu.sync_copy(dsts_hbm.at[pl.ds(w * cpw, cpw)], dall)

        def zrow(r, carry):
            stage[pl.ds(r * 16, 16)] = jnp.zeros((16,), jnp.float32)
            return carry

        lax.fori_loop(0, npt // 16, zrow, 0)
        pltpu.sync_copy(stage, accd.at[pl.ds(s * npt, npt)])
        for j in range(8):
            ones[pl.ds(j * 16, 16)] = jnp.ones((16,), jnp.float32)
        plsc.subcore_barrier()

        def body(k, carry):
            for j in range(8):
                pltpu.async_copy(ones.at[pl.ds(0, DCH)],
                                 accd.at[dall.at[k * 8 + j]], semd, add=True)
            for j in range(8):
                pltpu.make_async_copy(ones.at[pl.ds(0, DCH)],
                                      accd.at[dall.at[0]], semd).wait()
            return carry

        lax.fori_loop(0, cpw // 8, body, 0)
        plsc.subcore_barrier()
        pltpu.sync_copy(accd.at[pl.ds(s * npt, npt)], stage)
        pltpu.sync_copy(stage, out_hbm.at[pl.ds(c * n_pad + s * npt, npt)])

    return deg


@functools.lru_cache(maxsize=None)
def _make_unpool():
    nchunks = N2 // 80
    mesh = plsc.VectorSubcoreMesh(core_axis_name="c", subcore_axis_name="s")

    @functools.partial(
        pl.kernel,
        mesh=mesh,
        out_type=jax.ShapeDtypeStruct((N2, D), jnp.float32),
        scratch_types=[
            pltpu.VMEM((80,), jnp.int32),
            pltpu.VMEM((80, D), jnp.float32),
            pltpu.SemaphoreType.DMA,
        ],
    )
    def unpool(x_hbm, cols_hbm, u_hbm, cbuf, rows, sem):
        c = lax.axis_index("c")
        s = lax.axis_index("s")
        w = c * 16 + s
        nmine = (nchunks - w + 31) // 32

        def body(k, carry):
            rb = pl.multiple_of((w + k * 32) * 80, 16)
            pltpu.sync_copy(cols_hbm.at[pl.ds(rb, 80)], cbuf)
            pltpu.async_copy(x_hbm.at[cbuf], rows, sem).wait()
            pltpu.sync_copy(rows, u_hbm.at[pl.ds(rb, 80)])
            return carry

        lax.fori_loop(0, nmine, body, 0)

    return unpool


def _dis_body(deg2_ref, dis_ref, dis2n_ref):
    dsum = deg2_ref[0] + deg2_ref[1]
    good = dsum > 0
    dis = jnp.where(good, lax.rsqrt(jnp.where(good, dsum, 1.0)), 0.0)
    dis_ref[...] = dis
    dis2n_ref[...] = -(dis * dis)


def _make_dis(n):
    return pl.pallas_call(
        _dis_body,
        grid=(n // BN,),
        in_specs=[pl.BlockSpec((2, BN, 1), lambda i: (0, i, 0))],
        out_specs=[pl.BlockSpec((BN, 1), lambda i: (i, 0)),
                   pl.BlockSpec((BN, 1), lambda i: (i, 0))],
        out_shape=[jax.ShapeDtypeStruct((n, 1), jnp.float32),
                   jax.ShapeDtypeStruct((n, 1), jnp.float32)],
    )


def _scale_body(x_ref, s_ref, y_ref):
    y_ref[...] = x_ref[...] * s_ref[...]


def _make_scale(n, n_out=None):
    return pl.pallas_call(
        _scale_body,
        grid=(n // BN,),
        in_specs=[pl.BlockSpec((BN, D), lambda i: (i, 0)),
                  pl.BlockSpec((BN, 1), lambda i: (i, 0))],
        out_specs=pl.BlockSpec((BN, D), lambda i: (i, 0)),
        out_shape=jax.ShapeDtypeStruct((n_out or n, D), jnp.float32),
    )


def _out_body(with_stats, tx0_ref, s0_ref, s1_ref, dis_ref, wa_ref, wb_ref,
              wc_ref, b_ref, o_ref, *maybe_st):
    dis = dis_ref[...]
    tx0 = tx0_ref[...]
    tx1 = -(s0_ref[...] * dis)
    tx2 = 2.0 * (-(s1_ref[...] * dis)) - tx0
    o = jnp.dot(tx0, wa_ref[...], preferred_element_type=jnp.float32)
    o = o + jnp.dot(tx1, wb_ref[...], preferred_element_type=jnp.float32)
    o = o + jnp.dot(tx2, wc_ref[...], preferred_element_type=jnp.float32)
    o = o + b_ref[...]
    o_ref[...] = o
    if with_stats:
        st_ref = maybe_st[0]

        @pl.when(pl.program_id(0) == 0)
        def _init():
            st_ref[...] = jnp.zeros_like(st_ref)

        st_ref[...] += jnp.concatenate(
            [jnp.sum(o, axis=0, keepdims=True),
             jnp.sum(o * o, axis=0, keepdims=True)], axis=0)


def _make_out(n, with_stats):
    out_specs = [pl.BlockSpec((BN, D), lambda i: (i, 0))]
    out_shape = [jax.ShapeDtypeStruct((n, D), jnp.float32)]
    if with_stats:
        out_specs.append(pl.BlockSpec((2, D), lambda i: (0, 0)))
        out_shape.append(jax.ShapeDtypeStruct((2, D), jnp.float32))
    return pl.pallas_call(
        functools.partial(_out_body, with_stats),
        grid=(n // BN,),
        in_specs=[pl.BlockSpec((BN, D), lambda i: (i, 0)),
                  pl.BlockSpec((BN, D), lambda i: (i, 0)),
                  pl.BlockSpec((BN, D), lambda i: (i, 0)),
                  pl.BlockSpec((BN, 1), lambda i: (i, 0)),
                  pl.BlockSpec((D, D), lambda i: (0, 0)),
                  pl.BlockSpec((D, D), lambda i: (0, 0)),
                  pl.BlockSpec((D, D), lambda i: (0, 0)),
                  pl.BlockSpec((1, D), lambda i: (0, 0))],
        out_specs=out_specs if with_stats else out_specs[0],
        out_shape=out_shape if with_stats else out_shape[0],
    )


def _stats_body(x_ref, vals_ref, st_ref):
    x = x_ref[...] * vals_ref[...]

    @pl.when(pl.program_id(0) == 0)
    def _init():
        st_ref[...] = jnp.zeros_like(st_ref)

    st_ref[...] += jnp.concatenate(
        [jnp.sum(x, axis=0, keepdims=True),
         jnp.sum(x * x, axis=0, keepdims=True)], axis=0)


def _make_stats(n):
    return pl.pallas_call(
        _stats_body,
        grid=(n // BN,),
        in_specs=[pl.BlockSpec((BN, D), lambda i: (i, 0)),
                  pl.BlockSpec((BN, 1), lambda i: (i, 0))],
        out_specs=pl.BlockSpec((2, D), lambda i: (0, 0)),
        out_shape=jax.ShapeDtypeStruct((2, D), jnp.float32),
    )


def _bn_body(n, with_vals, with_g, *refs):
    refs = list(refs)
    x_ref = refs.pop(0)
    st_ref = refs.pop(0)
    gam_ref = refs.pop(0)
    bet_ref = refs.pop(0)
    vals_ref = refs.pop(0) if with_vals else None
    dis_ref = refs.pop(0) if with_g else None
    y_ref = refs.pop(0)
    g_ref = refs.pop(0) if with_g else None

    x = x_ref[...]
    if with_vals:
        x = x * vals_ref[...]
    inv_n = 1.0 / n
    mean = st_ref[0:1, :] * inv_n
    var = st_ref[1:2, :] * inv_n - mean * mean
    scale = lax.rsqrt(var + EPS) * gam_ref[...]
    y = (x - mean) * scale + bet_ref[...]
    y = jnp.where(y >= 0, y, NEG_SLOPE * y)
    y_ref[...] = y
    if with_g:
        g_ref[...] = y * dis_ref[...]


def _make_bn(n, with_vals, with_g):
    in_specs = [pl.BlockSpec((BN, D), lambda i: (i, 0)),
                pl.BlockSpec((2, D), lambda i: (0, 0)),
                pl.BlockSpec((1, D), lambda i: (0, 0)),
                pl.BlockSpec((1, D), lambda i: (0, 0))]
    if with_vals:
        in_specs.append(pl.BlockSpec((BN, 1), lambda i: (i, 0)))
    if with_g:
        in_specs.append(pl.BlockSpec((BN, 1), lambda i: (i, 0)))
    out_specs = [pl.BlockSpec((BN, D), lambda i: (i, 0))]
    out_shape = [jax.ShapeDtypeStruct((n, D), jnp.float32)]
    if with_g:
        out_specs.append(pl.BlockSpec((BN, D), lambda i: (i, 0)))
        out_shape.append(jax.ShapeDtypeStruct((n, D), jnp.float32))
    return pl.pallas_call(
        functools.partial(_bn_body, n, with_vals, with_g),
        grid=(n // BN,),
        in_specs=in_specs,
        out_specs=out_specs if with_g else out_specs[0],
        out_shape=out_shape if with_g else out_shape[0],
    )


_dis_n1 = _make_dis(N1)
_dis_n2 = _make_dis(N2)
_scale_n1w = _make_scale(N1, N2)
_scale_n2 = _make_scale(N2)
_out_n1 = _make_out(N1, False)
_out_n2 = _make_out(N2, True)
_stats_n2 = _make_stats(N2)
_bn_unpool = _make_bn(N2, True, True)
_bn_mid = _make_bn(N2, False, True)
_bn_final = _make_bn(N2, False, False)


def _local_dsts(dst, n_nodes):
    nh = n_nodes // 2
    spread = jnp.arange(E, dtype=jnp.int32) % NDUMMY
    parts = []
    for c in (0, 1):
        mine = (dst >= c * nh) & (dst < (c + 1) * nh)
        parts.append(jnp.where(mine, dst - c * nh, nh + spread))
    return jnp.concatenate(parts).reshape(2 * NCH, CH)


def kernel(input, edge_index1, edge_index2, unpool_rows, unpool_cols,
           unpool_vals, W1, b1, bn1_gamma, bn1_beta, W2, b2,
           bn2_gamma, bn2_beta):
    f32 = jnp.float32
    src1, dst1 = edge_index1[0], edge_index1[1]
    src2, dst2 = edge_index2[0], edge_index2[1]
    dl1 = _local_dsts(dst1, N2)
    dl2 = _local_dsts(dst2, N2)
    src1 = src1.reshape(NCH, CH)
    src2 = src2.reshape(NCH, CH)

    _prop_n2 = _make_prop(N2)
    _unpool = _make_unpool()
    deg1p = _make_deg(_pad16(N1))(dst1.reshape(DNCH, DCH))
    deg2p = _make_deg(_pad16(N2))(dst2.reshape(DNCH, DCH))
    deg1p = jnp.stack([deg1p[:N1], deg1p[_pad16(N1):_pad16(N1) + N1]])
    deg2p = jnp.stack([deg2p[:N2], deg2p[_pad16(N2):_pad16(N2) + N2]])
    dis1, dis2n1 = _dis_n1(deg1p.reshape(2, N1, 1))
    dis2, dis2n2 = _dis_n2(deg2p.reshape(2, N2, 1))

    g = _scale_n1w(input, dis1)
    s0 = _prop_n2(g, src1, dl1)
    g1 = _scale_n1w(s0, dis2n1)
    s1 = _prop_n2(g1, src1, dl1)
    x1 = _out_n1(input, s0, s1, dis1, W1[0], W1[1], W1[2], b1.reshape(1, D))

    u = _unpool(x1, unpool_cols)
    vals = unpool_vals.reshape(N2, 1)
    st = _stats_n2(u, vals)
    y, g = _bn_unpool(u, st, bn1_gamma.reshape(1, D), bn1_beta.reshape(1, D),
                      vals, dis2)

    for i in range(4):
        s0 = _prop_n2(g, src2, dl2)
        g1 = _scale_n2(s0, dis2n2)
        s1 = _prop_n2(g1, src2, dl2)
        xp, st = _out_n2(y, s0, s1, dis2, W2[i, 0], W2[i, 1], W2[i, 2],
                         b2[i].reshape(1, D))
        if i < 3:
            y, g = _bn_mid(xp, st, bn2_gamma[i].reshape(1, D),
                           bn2_beta[i].reshape(1, D), dis2)
        else:
            y = _bn_final(xp, st, bn2_gamma[i].reshape(1, D),
                          bn2_beta[i].reshape(1, D))
    return y
